# baseline (speedup 1.0000x reference)
import functools

import jax
import jax.numpy as jnp
from jax import lax
from jax.experimental import pallas as pl
from jax.experimental.pallas import tpu as pltpu
from jax.experimental.pallas import tpu_sc as plsc

N_VERTICES = 1000000
N_DIM = 32
BATCH = 16384

NC = 2
NS = 16
L = 16
NW = NC * NS

CV = 1024
NCHUNK = (N_VERTICES + CV - 1) // CV
LAST_CID = NCHUNK - 1
TAIL = 64
LAST_CV = N_VERTICES - LAST_CID * CV - TAIL
BASE_CPW = NCHUNK // NW
EXTRA = NCHUNK - BASE_CPW * NW
MAX_CPW = BASE_CPW + 1

CAP = 4096
EMB_W = 128
U_BASE = BATCH + 32
EMB_ROWS = 2 * U_BASE


def _ch0(w):
    return BASE_CPW * w + jnp.minimum(w, EXTRA)


def _emb_body(tableT_hbm, tail_hbm, pos_v_hbm, pos_u_hbm, emb_hbm,
              pv, pu, cbuf, hv, hm, chv, chm, rowbuf, sem_s, sem_c):
    wid = lax.axis_index("s") * NC + lax.axis_index("c")
    ch0 = _ch0(wid)
    ch1 = _ch0(wid + 1)
    iota = lax.iota(jnp.int32, L)

    pltpu.sync_copy(pos_v_hbm, pv)
    pltpu.sync_copy(pos_u_hbm, pu)

    def issue_chunk(cid, buf, valid):
        @pl.when(valid & (cid != LAST_CID))
        def _():
            pltpu.make_async_copy(
                tableT_hbm.at[:, pl.ds(cid * CV, CV)], cbuf.at[buf], sem_s
            ).start()

        @pl.when(valid & (cid == LAST_CID))
        def _():
            pltpu.make_async_copy(
                tableT_hbm.at[:, pl.ds(LAST_CID * CV, LAST_CV)],
                cbuf.at[buf, :, pl.ds(0, LAST_CV)], sem_s,
            ).start()

    def wait_chunk(cid, buf, valid):
        @pl.when(valid & (cid != LAST_CID))
        def _():
            pltpu.make_async_copy(
                tableT_hbm.at[:, pl.ds(0, CV)], cbuf.at[buf], sem_s
            ).wait()

        @pl.when(valid & (cid == LAST_CID))
        def _():
            pltpu.make_async_copy(
                tableT_hbm.at[:, pl.ds(0, LAST_CV)],
                cbuf.at[buf, :, pl.ds(0, LAST_CV)], sem_s,
            ).wait()
            pltpu.sync_copy(tail_hbm, cbuf.at[buf, :, pl.ds(LAST_CV, 128)])

    def scan_pass(pos_ref, meta_base, jbase, jstart, carry):
        def body(j, c):
            ptr_st, stopped, jstop = c
            vec = pos_ref[pl.ds(j * L, L)]
            t = lax.shift_right_logical(vec, 10)
            m = (t >= ch0) & (t < ch1)
            gj = jbase + j
            active = (gj >= jstart) & jnp.logical_not(stopped)
            keep0 = m & jnp.full((L,), active, jnp.bool_)
            pk = plsc.all_reduce_population_count(keep0)[0]
            fits = ptr_st + pk <= CAP
            keep = keep0 & jnp.full((L,), fits, jnp.bool_)
            plsc.store_compressed(hv.at[pl.ds(ptr_st, L)], vec, mask=keep)
            meta = meta_base + j * L + iota
            plsc.store_compressed(hm.at[pl.ds(ptr_st, L)], meta, mask=keep)
            new_stop = active & jnp.logical_not(fits)
            jstop = jnp.where(new_stop & jnp.logical_not(stopped), gj, jstop)
            return (ptr_st + jnp.where(fits, pk, 0),
                    stopped | new_stop, jstop)

        return lax.fori_loop(0, BATCH // L, body, carry)

    NVREG = 2 * (BATCH // L)

    def round_body(jstart):
        st = scan_pass(pv, 0, 0, jstart,
                       (jnp.int32(0), jnp.bool_(False), jnp.int32(NVREG)))
        st = scan_pass(pu, U_BASE, BATCH // L, jstart, st)
        nh, _stopped, jstop = st

        issue_chunk(ch0, 0, True)
        issue_chunk(ch0 + 1, 1, True)

        def chunk_body(c, carry):
            cid = ch0 + c
            valid = cid < ch1
            buf = c % 2
            wait_chunk(cid, buf, valid)

            def rescan(k, cptr):
                hvv = hv[pl.ds(k * L, L)]
                hmv = hm[pl.ds(k * L, L)]
                live = (k * L + iota) < nh
                mm = live & (lax.shift_right_logical(hvv, 10) == cid)
                pk = plsc.all_reduce_population_count(mm)[0]
                plsc.store_compressed(chv.at[pl.ds(cptr, L)], hvv, mask=mm)
                plsc.store_compressed(chm.at[pl.ds(cptr, L)], hmv, mask=mm)
                return cptr + pk

            cptr = lax.fori_loop(0, (nh + L - 1) // L, rescan, jnp.int32(0))

            def extract(k2, xcarry):
                b = k2 % 4
                cv16 = chv[pl.ds(k2 * L, L)]
                cm16 = chm[pl.ds(k2 * L, L)]
                rem = cptr - k2 * L
                lanevalid = iota < rem
                vloc = jnp.clip(cv16 - cid * CV, 0, CV - 1)
                midx = jnp.where(lanevalid, cm16, BATCH + iota)

                @pl.when(k2 >= 4)
                def _():
                    pltpu.make_async_copy(
                        emb_hbm.at[pl.ds(0, L)], rowbuf.at[0], sem_c
                    ).wait()

                bsplat = jnp.full((L,), buf, jnp.int32)
                rsplat = jnp.full((L,), b, jnp.int32)
                for d in range(N_DIM):
                    dsplat = jnp.full((L,), d, jnp.int32)
                    vals = plsc.load_gather(cbuf, [bsplat, dsplat, vloc])
                    plsc.store_scatter(rowbuf, [rsplat, iota, dsplat], vals)
                pltpu.make_async_copy(rowbuf.at[b], emb_hbm.at[midx], sem_c).start()
                return xcarry

            ngc = (cptr + L - 1) // L
            lax.fori_loop(0, ngc, extract, 0)

            def drain(dk, dcarry):
                pltpu.make_async_copy(
                    emb_hbm.at[pl.ds(0, L)], rowbuf.at[0], sem_c
                ).wait()
                return dcarry

            lax.fori_loop(0, jnp.minimum(ngc, 4), drain, 0)

            issue_chunk(cid + 2, buf, cid + 2 < ch1)
            return carry

        lax.fori_loop(0, MAX_CPW, chunk_body, 0)
        return jstop

    lax.while_loop(lambda jstart: jstart < NVREG, round_body, jnp.int32(0))


@jax.jit
def _sc_emb(tableT, tail, pos_v, pos_u):
    mesh = plsc.VectorSubcoreMesh(core_axis_name="c", subcore_axis_name="s")
    k = functools.partial(
        pl.kernel,
        mesh=mesh,
        out_type=jax.ShapeDtypeStruct((EMB_ROWS, EMB_W), jnp.float32),
        scratch_types=[
            pltpu.VMEM((BATCH,), jnp.int32),
            pltpu.VMEM((BATCH,), jnp.int32),
            pltpu.VMEM((2, N_DIM, CV), jnp.float32),
            pltpu.VMEM((CAP + L,), jnp.int32),
            pltpu.VMEM((CAP + L,), jnp.int32),
            pltpu.VMEM((CAP + L,), jnp.int32),
            pltpu.VMEM((CAP + L,), jnp.int32),
            pltpu.VMEM((4, L, EMB_W), jnp.float32),
            pltpu.SemaphoreType.DMA,
            pltpu.SemaphoreType.DMA,
        ],
        compiler_params=pltpu.CompilerParams(needs_layout_passes=False),
    )(_emb_body)
    return k(tableT, tail, pos_v, pos_u)


BPW = BATCH // NW
SB = 256


def _dot_body(emb_hbm, score_hbm, rv, ru, score_v, sem):
    wid = lax.axis_index("s") * NC + lax.axis_index("c")
    base = wid * BPW
    lane = lax.iota(jnp.int32, L)

    def sub(s, carry):
        row0 = base + s * SB
        cp_v = pltpu.async_copy(emb_hbm.at[pl.ds(row0, SB)], rv, sem)
        cp_u = pltpu.async_copy(emb_hbm.at[pl.ds(U_BASE + row0, SB)], ru, sem)
        cp_v.wait()
        cp_u.wait()

        def group(g, c):
            acc = jnp.zeros((L,), jnp.float32)
            for i in range(L):
                r = g * L + i
                prod = (rv[r, pl.ds(0, L)] * ru[r, pl.ds(0, L)]
                        + rv[r, pl.ds(L, L)] * ru[r, pl.ds(L, L)])
                acc = jnp.where(lane == i, jnp.sum(prod), acc)
            score_v[pl.ds(g * L, L)] = acc
            return c

        lax.fori_loop(0, SB // L, group, 0)
        pltpu.sync_copy(score_v, score_hbm.at[pl.ds(row0, SB)])
        return carry

    lax.fori_loop(0, BPW // SB, sub, 0)


@jax.jit
def _sc_dot(emb):
    mesh = plsc.VectorSubcoreMesh(core_axis_name="c", subcore_axis_name="s")
    k = functools.partial(
        pl.kernel,
        mesh=mesh,
        out_type=jax.ShapeDtypeStruct((BATCH,), jnp.float32),
        scratch_types=[
            pltpu.VMEM((SB, EMB_W), jnp.float32),
            pltpu.VMEM((SB, EMB_W), jnp.float32),
            pltpu.VMEM((SB,), jnp.float32),
            pltpu.SemaphoreType.DMA,
        ],
        compiler_params=pltpu.CompilerParams(needs_layout_passes=False),
    )(_dot_body)
    return k(emb)


def _tc_loss_body(score_ref, w_ref, out_ref):
    s = score_ref[...]
    w = w_ref[...]
    loss = (jnp.maximum(-s, 0.0) + jnp.log1p(jnp.exp(-jnp.abs(s)))) * w
    out_ref[...] = jnp.sum(loss).reshape(1, 1)


@jax.jit
def _tc_loss(score, weights):
    out = pl.pallas_call(
        _tc_loss_body,
        out_shape=jax.ShapeDtypeStruct((1, 1), jnp.float32),
    )(score.reshape(128, 128), weights.reshape(128, 128))
    return out[0, 0]


def kernel(pos_v, pos_u, weights, table):
    pos_v = jnp.asarray(pos_v, jnp.int32)
    pos_u = jnp.asarray(pos_u, jnp.int32)
    tableT = table.T
    tail = jnp.pad(tableT[:, N_VERTICES - TAIL:], ((0, 0), (0, 128 - TAIL)))
    emb = _sc_emb(tableT, tail, pos_v, pos_u)
    score = _sc_dot(emb)
    return _tc_loss(score, weights)


# speedup vs baseline: 1.0680x; 1.0680x over previous
import functools

import jax
import jax.numpy as jnp
from jax import lax
from jax.experimental import pallas as pl
from jax.experimental.pallas import tpu as pltpu
from jax.experimental.pallas import tpu_sc as plsc

N_VERTICES = 1000000
N_DIM = 32
BATCH = 16384

NC = 2
NS = 16
L = 16
NW = NC * NS

CV = 1024
NCHUNK = (N_VERTICES + CV - 1) // CV
LAST_CID = NCHUNK - 1
TAIL = 64
LAST_CV = N_VERTICES - LAST_CID * CV - TAIL
BASE_CPW = NCHUNK // NW
EXTRA = NCHUNK - BASE_CPW * NW
MAX_CPW = BASE_CPW + 1

CAP = 4096
EMB_W = 128
U_BASE = BATCH + 32
EMB_ROWS = 2 * U_BASE


def _ch0(w):
    return BASE_CPW * w + jnp.minimum(w, EXTRA)


def _emb_body(tableT_hbm, tail_hbm, pos_v_hbm, pos_u_hbm, emb_hbm,
              pv, pu, cbuf, hv, hm, chv, chm, rowbuf, sem_s, sem_c):
    wid = lax.axis_index("s") * NC + lax.axis_index("c")
    ch0 = _ch0(wid)
    ch1 = _ch0(wid + 1)
    iota = lax.iota(jnp.int32, L)

    pltpu.sync_copy(pos_v_hbm, pv)
    pltpu.sync_copy(pos_u_hbm, pu)

    def issue_chunk(cid, buf, valid):
        @pl.when(valid & (cid != LAST_CID))
        def _():
            pltpu.make_async_copy(
                tableT_hbm.at[:, pl.ds(cid * CV, CV)], cbuf.at[buf], sem_s
            ).start()

        @pl.when(valid & (cid == LAST_CID))
        def _():
            pltpu.make_async_copy(
                tableT_hbm.at[:, pl.ds(LAST_CID * CV, LAST_CV)],
                cbuf.at[buf, :, pl.ds(0, LAST_CV)], sem_s,
            ).start()

    def wait_chunk(cid, buf, valid):
        @pl.when(valid & (cid != LAST_CID))
        def _():
            pltpu.make_async_copy(
                tableT_hbm.at[:, pl.ds(0, CV)], cbuf.at[buf], sem_s
            ).wait()

        @pl.when(valid & (cid == LAST_CID))
        def _():
            pltpu.make_async_copy(
                tableT_hbm.at[:, pl.ds(0, LAST_CV)],
                cbuf.at[buf, :, pl.ds(0, LAST_CV)], sem_s,
            ).wait()
            pltpu.sync_copy(tail_hbm, cbuf.at[buf, :, pl.ds(LAST_CV, 128)])

    def scan_pass(pos_ref, meta_base, start, carry):
        def body(j, c):
            ptr_all, ptr_st = c
            vec = pos_ref[pl.ds(j * L, L)]
            t = lax.shift_right_logical(vec, 10)
            m = (t >= ch0) & (t < ch1)
            m01 = m.astype(jnp.int32)
            cs = plsc.cumsum(m01)
            n_here = cs[L - 1]
            rank = ptr_all + cs
            keep = m & (rank > start) & (rank <= start + CAP)
            pk = plsc.all_reduce_population_count(keep)[0]
            plsc.store_compressed(hv.at[pl.ds(ptr_st, L)], vec, mask=keep)
            meta = meta_base + j * L + iota
            plsc.store_compressed(hm.at[pl.ds(ptr_st, L)], meta, mask=keep)
            return (ptr_all + n_here, ptr_st + pk)

        return lax.fori_loop(0, BATCH // L, body, carry)

    def round_body(state):
        r, _total = state
        start = r * CAP
        ptr_all, nh = scan_pass(pv, 0, start, (jnp.int32(0), jnp.int32(0)))
        ptr_all, nh = scan_pass(pu, U_BASE, start, (ptr_all, nh))

        issue_chunk(ch0, 0, True)
        issue_chunk(ch0 + 1, 1, True)

        def chunk_body(c, carry):
            cid = ch0 + c
            valid = cid < ch1
            buf = c % 2
            wait_chunk(cid, buf, valid)

            def rescan(k, cptr):
                hvv = hv[pl.ds(k * L, L)]
                hmv = hm[pl.ds(k * L, L)]
                live = (k * L + iota) < nh
                mm = live & (lax.shift_right_logical(hvv, 10) == cid)
                pk = plsc.all_reduce_population_count(mm)[0]
                plsc.store_compressed(chv.at[pl.ds(cptr, L)], hvv, mask=mm)
                plsc.store_compressed(chm.at[pl.ds(cptr, L)], hmv, mask=mm)
                return cptr + pk

            cptr = lax.fori_loop(0, (nh + L - 1) // L, rescan, jnp.int32(0))

            def extract(k2, xcarry):
                b = k2 % 4
                cv16 = chv[pl.ds(k2 * L, L)]
                cm16 = chm[pl.ds(k2 * L, L)]
                rem = cptr - k2 * L
                lanevalid = iota < rem
                vloc = jnp.clip(cv16 - cid * CV, 0, CV - 1)
                midx = jnp.where(lanevalid, cm16, BATCH + iota)

                @pl.when(k2 >= 4)
                def _():
                    pltpu.make_async_copy(
                        emb_hbm.at[pl.ds(0, L)], rowbuf.at[0], sem_c
                    ).wait()

                bsplat = jnp.full((L,), buf, jnp.int32)
                for i in range(L):
                    vl = jnp.full((L,), vloc[i], jnp.int32)
                    lo = plsc.load_gather(cbuf, [bsplat, iota, vl])
                    hi = plsc.load_gather(cbuf, [bsplat, iota + L, vl])
                    rowbuf[b, i, pl.ds(0, L)] = lo
                    rowbuf[b, i, pl.ds(L, L)] = hi
                pltpu.make_async_copy(rowbuf.at[b], emb_hbm.at[midx], sem_c).start()
                return xcarry

            ngc = (cptr + L - 1) // L
            lax.fori_loop(0, ngc, extract, 0)

            def drain(dk, dcarry):
                pltpu.make_async_copy(
                    emb_hbm.at[pl.ds(0, L)], rowbuf.at[0], sem_c
                ).wait()
                return dcarry

            lax.fori_loop(0, jnp.minimum(ngc, 4), drain, 0)

            issue_chunk(cid + 2, buf, cid + 2 < ch1)
            return carry

        lax.fori_loop(0, MAX_CPW, chunk_body, 0)
        return (r + 1, ptr_all)

    def round_cond(state):
        r, total = state
        return r * CAP < total

    lax.while_loop(round_cond, round_body, (jnp.int32(0), jnp.int32(1)))


@jax.jit
def _sc_emb(tableT, tail, pos_v, pos_u):
    mesh = plsc.VectorSubcoreMesh(core_axis_name="c", subcore_axis_name="s")
    k = functools.partial(
        pl.kernel,
        mesh=mesh,
        out_type=jax.ShapeDtypeStruct((EMB_ROWS, EMB_W), jnp.float32),
        scratch_types=[
            pltpu.VMEM((BATCH,), jnp.int32),
            pltpu.VMEM((BATCH,), jnp.int32),
            pltpu.VMEM((2, N_DIM, CV), jnp.float32),
            pltpu.VMEM((CAP + L,), jnp.int32),
            pltpu.VMEM((CAP + L,), jnp.int32),
            pltpu.VMEM((CAP + L,), jnp.int32),
            pltpu.VMEM((CAP + L,), jnp.int32),
            pltpu.VMEM((4, L, EMB_W), jnp.float32),
            pltpu.SemaphoreType.DMA,
            pltpu.SemaphoreType.DMA,
        ],
        compiler_params=pltpu.CompilerParams(needs_layout_passes=False),
    )(_emb_body)
    return k(tableT, tail, pos_v, pos_u)


BPW = BATCH // NW
SB = 256


def _dot_body(emb_hbm, score_hbm, rv, ru, score_v, sem):
    wid = lax.axis_index("s") * NC + lax.axis_index("c")
    base = wid * BPW
    lane = lax.iota(jnp.int32, L)

    def sub(s, carry):
        row0 = base + s * SB
        cp_v = pltpu.async_copy(emb_hbm.at[pl.ds(row0, SB)], rv, sem)
        cp_u = pltpu.async_copy(emb_hbm.at[pl.ds(U_BASE + row0, SB)], ru, sem)
        cp_v.wait()
        cp_u.wait()

        def group(g, c):
            acc = jnp.zeros((L,), jnp.float32)
            for i in range(L):
                r = g * L + i
                prod = (rv[r, pl.ds(0, L)] * ru[r, pl.ds(0, L)]
                        + rv[r, pl.ds(L, L)] * ru[r, pl.ds(L, L)])
                acc = jnp.where(lane == i, jnp.sum(prod), acc)
            score_v[pl.ds(g * L, L)] = acc
            return c

        lax.fori_loop(0, SB // L, group, 0)
        pltpu.sync_copy(score_v, score_hbm.at[pl.ds(row0, SB)])
        return carry

    lax.fori_loop(0, BPW // SB, sub, 0)


@jax.jit
def _sc_dot(emb):
    mesh = plsc.VectorSubcoreMesh(core_axis_name="c", subcore_axis_name="s")
    k = functools.partial(
        pl.kernel,
        mesh=mesh,
        out_type=jax.ShapeDtypeStruct((BATCH,), jnp.float32),
        scratch_types=[
            pltpu.VMEM((SB, EMB_W), jnp.float32),
            pltpu.VMEM((SB, EMB_W), jnp.float32),
            pltpu.VMEM((SB,), jnp.float32),
            pltpu.SemaphoreType.DMA,
        ],
        compiler_params=pltpu.CompilerParams(needs_layout_passes=False),
    )(_dot_body)
    return k(emb)


def _tc_loss_body(score_ref, w_ref, out_ref):
    s = score_ref[...]
    w = w_ref[...]
    loss = (jnp.maximum(-s, 0.0) + jnp.log1p(jnp.exp(-jnp.abs(s)))) * w
    out_ref[...] = jnp.sum(loss).reshape(1, 1)


@jax.jit
def _tc_loss(score, weights):
    out = pl.pallas_call(
        _tc_loss_body,
        out_shape=jax.ShapeDtypeStruct((1, 1), jnp.float32),
    )(score.reshape(128, 128), weights.reshape(128, 128))
    return out[0, 0]


def kernel(pos_v, pos_u, weights, table):
    pos_v = jnp.asarray(pos_v, jnp.int32)
    pos_u = jnp.asarray(pos_u, jnp.int32)
    tableT = table.T
    tail = jnp.pad(tableT[:, N_VERTICES - TAIL:], ((0, 0), (0, 128 - TAIL)))
    emb = _sc_emb(tableT, tail, pos_v, pos_u)
    score = _sc_dot(emb)
    return _tc_loss(score, weights)


# speedup vs baseline: 1.0868x; 1.0176x over previous
import functools

import jax
import jax.numpy as jnp
from jax import lax
from jax.experimental import pallas as pl
from jax.experimental.pallas import tpu as pltpu
from jax.experimental.pallas import tpu_sc as plsc

N_VERTICES = 1000000
N_DIM = 32
BATCH = 16384

NC = 2
NS = 16
L = 16
NW = NC * NS

CV = 1024
NCHUNK = (N_VERTICES + CV - 1) // CV
LAST_CID = NCHUNK - 1
TAIL = 64
LAST_CV = N_VERTICES - LAST_CID * CV - TAIL
BASE_CPW = NCHUNK // NW
EXTRA = NCHUNK - BASE_CPW * NW
MAX_CPW = BASE_CPW + 1

CAP = 2816
EMB_W = 128
U_BASE = BATCH + 32
EMB_ROWS = 2 * U_BASE


def _ch0(w):
    return BASE_CPW * w + jnp.minimum(w, EXTRA)


def _emb_body(tableT_hbm, tail_hbm, pos_v_hbm, pos_u_hbm, emb_hbm,
              pp, cbuf, hv, hm, chv, chm, rowbuf, sem_s, sem_c):
    wid = lax.axis_index("s") * NC + lax.axis_index("c")
    ch0 = _ch0(wid)
    ch1 = _ch0(wid + 1)
    iota = lax.iota(jnp.int32, L)

    def issue_chunk(cid, buf, valid):
        @pl.when(valid & (cid != LAST_CID))
        def _():
            pltpu.make_async_copy(
                tableT_hbm.at[:, pl.ds(cid * CV, CV)], cbuf.at[buf], sem_s
            ).start()

        @pl.when(valid & (cid == LAST_CID))
        def _():
            pltpu.make_async_copy(
                tableT_hbm.at[:, pl.ds(LAST_CID * CV, LAST_CV)],
                cbuf.at[buf, :, pl.ds(0, LAST_CV)], sem_s,
            ).start()

    def wait_chunk(cid, buf, valid):
        @pl.when(valid & (cid != LAST_CID))
        def _():
            pltpu.make_async_copy(
                tableT_hbm.at[:, pl.ds(0, CV)], cbuf.at[buf], sem_s
            ).wait()

        @pl.when(valid & (cid == LAST_CID))
        def _():
            pltpu.make_async_copy(
                tableT_hbm.at[:, pl.ds(0, LAST_CV)],
                cbuf.at[buf, :, pl.ds(0, LAST_CV)], sem_s,
            ).wait()
            pltpu.sync_copy(tail_hbm, cbuf.at[buf, :, pl.ds(LAST_CV, 128)])

    def scan_pass(pos_ref, meta_base, start, carry):
        def body(j, c):
            ptr_all, ptr_st = c
            vec = pos_ref[pl.ds(j * L, L)]
            t = lax.shift_right_logical(vec, 10)
            m = (t >= ch0) & (t < ch1)
            m01 = m.astype(jnp.int32)
            cs = plsc.cumsum(m01)
            n_here = cs[L - 1]
            rank = ptr_all + cs
            keep = m & (rank > start) & (rank <= start + CAP)
            pk = plsc.all_reduce_population_count(keep)[0]
            plsc.store_compressed(hv.at[pl.ds(ptr_st, L)], vec, mask=keep)
            meta = meta_base + j * L + iota
            plsc.store_compressed(hm.at[pl.ds(ptr_st, L)], meta, mask=keep)
            return (ptr_all + n_here, ptr_st + pk)

        return lax.fori_loop(0, BATCH // L, body, carry)

    def round_body(state):
        r, _total = state
        start = r * CAP
        pltpu.sync_copy(pos_v_hbm, pp)
        ptr_all, nh = scan_pass(pp, 0, start, (jnp.int32(0), jnp.int32(0)))
        pltpu.sync_copy(pos_u_hbm, pp)
        ptr_all, nh = scan_pass(pp, U_BASE, start, (ptr_all, nh))

        issue_chunk(ch0, 0, True)
        issue_chunk(ch0 + 1, 1, True)

        def chunk_body(c, carry):
            cid = ch0 + c
            valid = cid < ch1
            buf = c % 3
            wait_chunk(cid, buf, valid)
            issue_chunk(cid + 2, (c + 2) % 3, cid + 2 < ch1)

            def rescan(k, cptr):
                hvv = hv[pl.ds(k * L, L)]
                hmv = hm[pl.ds(k * L, L)]
                live = (k * L + iota) < nh
                mm = live & (lax.shift_right_logical(hvv, 10) == cid)
                pk = plsc.all_reduce_population_count(mm)[0]
                plsc.store_compressed(chv.at[pl.ds(cptr, L)], hvv, mask=mm)
                plsc.store_compressed(chm.at[pl.ds(cptr, L)], hmv, mask=mm)
                return cptr + pk

            cptr = lax.fori_loop(0, (nh + L - 1) // L, rescan, jnp.int32(0))

            def extract(k2, xcarry):
                b = k2 % 2
                cv16 = chv[pl.ds(k2 * L, L)]
                cm16 = chm[pl.ds(k2 * L, L)]
                rem = cptr - k2 * L
                lanevalid = iota < rem
                vloc = jnp.clip(cv16 - cid * CV, 0, CV - 1)
                midx = jnp.where(lanevalid, cm16, BATCH + iota)

                @pl.when(k2 >= 2)
                def _():
                    pltpu.make_async_copy(
                        emb_hbm.at[pl.ds(0, L)], rowbuf.at[0], sem_c
                    ).wait()

                bsplat = jnp.full((L,), buf, jnp.int32)
                for i in range(L):
                    vl = jnp.full((L,), vloc[i], jnp.int32)
                    lo = plsc.load_gather(cbuf, [bsplat, iota, vl])
                    hi = plsc.load_gather(cbuf, [bsplat, iota + L, vl])
                    rowbuf[b, i, pl.ds(0, L)] = lo
                    rowbuf[b, i, pl.ds(L, L)] = hi
                pltpu.make_async_copy(rowbuf.at[b], emb_hbm.at[midx], sem_c).start()
                return xcarry

            ngc = (cptr + L - 1) // L
            lax.fori_loop(0, ngc, extract, 0)

            def drain(dk, dcarry):
                pltpu.make_async_copy(
                    emb_hbm.at[pl.ds(0, L)], rowbuf.at[0], sem_c
                ).wait()
                return dcarry

            lax.fori_loop(0, jnp.minimum(ngc, 2), drain, 0)
            return carry

        lax.fori_loop(0, MAX_CPW, chunk_body, 0)
        return (r + 1, ptr_all)

    def round_cond(state):
        r, total = state
        return r * CAP < total

    lax.while_loop(round_cond, round_body, (jnp.int32(0), jnp.int32(1)))


@jax.jit
def _sc_emb(tableT, tail, pos_v, pos_u):
    mesh = plsc.VectorSubcoreMesh(core_axis_name="c", subcore_axis_name="s")
    k = functools.partial(
        pl.kernel,
        mesh=mesh,
        out_type=jax.ShapeDtypeStruct((EMB_ROWS, EMB_W), jnp.float32),
        scratch_types=[
            pltpu.VMEM((BATCH,), jnp.int32),
            pltpu.VMEM((3, N_DIM, CV), jnp.float32),
            pltpu.VMEM((CAP + L,), jnp.int32),
            pltpu.VMEM((CAP + L,), jnp.int32),
            pltpu.VMEM((CAP + L,), jnp.int32),
            pltpu.VMEM((CAP + L,), jnp.int32),
            pltpu.VMEM((2, L, EMB_W), jnp.float32),
            pltpu.SemaphoreType.DMA,
            pltpu.SemaphoreType.DMA,
        ],
        compiler_params=pltpu.CompilerParams(needs_layout_passes=False),
    )(_emb_body)
    return k(tableT, tail, pos_v, pos_u)


BPW = BATCH // NW
SB = 256


def _dot_body(emb_hbm, score_hbm, rv, ru, score_v, sem):
    wid = lax.axis_index("s") * NC + lax.axis_index("c")
    base = wid * BPW
    lane = lax.iota(jnp.int32, L)

    def sub(s, carry):
        row0 = base + s * SB
        cp_v = pltpu.async_copy(emb_hbm.at[pl.ds(row0, SB)], rv, sem)
        cp_u = pltpu.async_copy(emb_hbm.at[pl.ds(U_BASE + row0, SB)], ru, sem)
        cp_v.wait()
        cp_u.wait()

        def group(g, c):
            acc = jnp.zeros((L,), jnp.float32)
            for i in range(L):
                r = g * L + i
                prod = (rv[r, pl.ds(0, L)] * ru[r, pl.ds(0, L)]
                        + rv[r, pl.ds(L, L)] * ru[r, pl.ds(L, L)])
                acc = jnp.where(lane == i, jnp.sum(prod), acc)
            score_v[pl.ds(g * L, L)] = acc
            return c

        lax.fori_loop(0, SB // L, group, 0)
        pltpu.sync_copy(score_v, score_hbm.at[pl.ds(row0, SB)])
        return carry

    lax.fori_loop(0, BPW // SB, sub, 0)


@jax.jit
def _sc_dot(emb):
    mesh = plsc.VectorSubcoreMesh(core_axis_name="c", subcore_axis_name="s")
    k = functools.partial(
        pl.kernel,
        mesh=mesh,
        out_type=jax.ShapeDtypeStruct((BATCH,), jnp.float32),
        scratch_types=[
            pltpu.VMEM((SB, EMB_W), jnp.float32),
            pltpu.VMEM((SB, EMB_W), jnp.float32),
            pltpu.VMEM((SB,), jnp.float32),
            pltpu.SemaphoreType.DMA,
        ],
        compiler_params=pltpu.CompilerParams(needs_layout_passes=False),
    )(_dot_body)
    return k(emb)


def _tc_loss_body(score_ref, w_ref, out_ref):
    s = score_ref[...]
    w = w_ref[...]
    loss = (jnp.maximum(-s, 0.0) + jnp.log1p(jnp.exp(-jnp.abs(s)))) * w
    out_ref[...] = jnp.sum(loss).reshape(1, 1)


@jax.jit
def _tc_loss(score, weights):
    out = pl.pallas_call(
        _tc_loss_body,
        out_shape=jax.ShapeDtypeStruct((1, 1), jnp.float32),
    )(score.reshape(128, 128), weights.reshape(128, 128))
    return out[0, 0]


def kernel(pos_v, pos_u, weights, table):
    pos_v = jnp.asarray(pos_v, jnp.int32)
    pos_u = jnp.asarray(pos_u, jnp.int32)
    tableT = table.T
    tail = jnp.pad(tableT[:, N_VERTICES - TAIL:], ((0, 0), (0, 128 - TAIL)))
    emb = _sc_emb(tableT, tail, pos_v, pos_u)
    score = _sc_dot(emb)
    return _tc_loss(score, weights)
